# Initial kernel scaffold; baseline (speedup 1.0000x reference)
import jax
import jax.numpy as jnp
from jax.experimental import pallas as pl


def kernel(x, edge_index, edge_attr, params):
    raise NotImplementedError("write your pallas kernel here")


# speedup vs baseline: 2.8838x; 2.8838x over previous
import functools

import jax
import jax.numpy as jnp
from jax import lax
from jax.experimental import pallas as pl
from jax.experimental.pallas import tpu as pltpu
from jax.experimental.pallas import tpu_sc as plsc

_N = 10000
_E = 320000
_D = 64
_L = 5

_NC = 2
_NS = 16
_LANE = 16
_NW = _NC * _NS
_EPW = _E // _NW
_CHUNK = 80
_NCHUNK = _EPW // _CHUNK
_IB = 5
_NBLK = _NCHUNK // _IB
_RPT = _N // _NS
_ZR = 25
_OUTR = 624
_BE2 = 2000


def _prelu(v, a):
    return jnp.where(v >= 0, v, a * v)


def _dot(a, b):
    return jnp.dot(a, b, preferred_element_type=jnp.float32)


def _node_enc_body(x_ref, w1, b1, a1, w2, b2, w1ij, be,
                   h_ref, t_ref):
    m = _prelu(_dot(x_ref[...], w1[...]) + b1[...], a1[0, 0])
    h = _dot(m, w2[...]) + b2[...]
    h_ref[...] = h
    t_ref[...] = _dot(h, w1ij[...]) + be[...]


def _edge_enc_body(ea_ref, w1, b1, a1, w2, b2, wst, o0, o1, o2, o3, o4):
    m = _prelu(_dot(ea_ref[...], w1[...]) + b1[...], a1[0, 0])
    e = _dot(m, w2[...]) + b2[...]
    for l, o in enumerate((o0, o1, o2, o3, o4)):
        o[...] = _dot(e, wst[l])


def _update_core(h_ref, s_ref, w2, b2, v1h, v1a, c1, an, v2, c2):
    h = h_ref[...]
    ssum = s_ref[0] + s_ref[1]
    s = ssum[:, :_D]
    deg = ssum[:, _D:_D + 1]
    aggr = _dot(s, w2[...]) + deg * b2[...]
    z = _prelu(_dot(h, v1h[...]) + _dot(aggr, v1a[...]) + c1[...], an[0, 0])
    return h + _dot(z, v2[...]) + c2[...]


def _update_body(h_ref, s_ref, w2, b2, v1h, v1a, c1, an, v2, c2,
                 nw1ij, nbe, hn_ref, t_ref):
    hn = _update_core(h_ref, s_ref, w2, b2, v1h, v1a, c1, an, v2, c2)
    hn_ref[...] = hn
    t_ref[...] = _dot(hn, nw1ij[...]) + nbe[...]


def _final_body(h_ref, s_ref, w2, b2, v1h, v1a, c1, an, v2, c2,
                dw1, db1, da, dw2, db2, out_ref):
    hn = _update_core(h_ref, s_ref, w2, b2, v1h, v1a, c1, an, v2, c2)
    t = _prelu(_dot(hn, dw1[...]) + db1[...], da[0, 0])
    out_ref[...] = _dot(t, dw2[...]) + db2[...]


def _seg_body(dst_hbm, src_hbm, t_hbm, ec_hbm, a_hbm, s_out,
              idxd, idxs, ti_v, tj_v, ec_v, z_v, a_v,
              s_sh, sem_e, sem_i, sem_j):
    cid = lax.axis_index("c")
    sid = lax.axis_index("s")
    wid = sid * _NC + cid

    pltpu.sync_copy(a_hbm, a_v)
    alpha = a_v[...]

    zero = jnp.zeros((_LANE,), jnp.float32)
    one = jnp.ones((_LANE,), jnp.float32)

    def zrow(r, _):
        for c4 in range(2 * _D // _LANE):
            z_v[r, pl.ds(c4 * _LANE, _LANE)] = zero
        return 0

    lax.fori_loop(0, _ZR, zrow, 0)
    for j in range(_RPT // _ZR):
        pltpu.sync_copy(z_v, s_sh.at[pl.ds(sid * _RPT + j * _ZR, _ZR)])

    plsc.subcore_barrier()

    def blk(b, _):
        pltpu.sync_copy(dst_hbm.at[wid, b], idxd)
        pltpu.sync_copy(src_hbm.at[wid, b], idxs)
        for u in range(_IB):
            c = b * _IB + u
            base = pl.multiple_of(wid * _EPW + c * _CHUNK, _CHUNK)
            cp_e = pltpu.async_copy(
                ec_hbm.at[pl.ds(pl.multiple_of(base // 2, _CHUNK // 2),
                                _CHUNK // 2)], ec_v, sem_e)
            cp_i = pltpu.async_copy(t_hbm.at[idxd.at[u]], ti_v, sem_i)
            cp_j = pltpu.async_copy(t_hbm.at[idxs.at[u]], tj_v, sem_j)
            cp_e.wait()
            cp_i.wait()
            cp_j.wait()

            def cw(r, _):
                eh = (r % 2) * _D
                for c4 in range(_D // _LANE):
                    sl = pl.ds(c4 * _LANE, _LANE)
                    v = (ti_v[r, sl]
                         + tj_v[r, pl.ds(_D + c4 * _LANE, _LANE)]
                         + ec_v[r // 2, pl.ds(eh + c4 * _LANE, _LANE)])
                    ti_v[r, sl] = jnp.where(v >= 0, v, alpha * v)
                for c4 in range(_D // _LANE, 2 * _D // _LANE):
                    ti_v[r, pl.ds(c4 * _LANE, _LANE)] = (
                        one if c4 == _D // _LANE else zero)
                return 0

            lax.fori_loop(0, _CHUNK, cw, 0)
            pltpu.sync_copy(ti_v, s_sh.at[idxd.at[u]], add=True)
        return 0

    lax.fori_loop(0, _NBLK, blk, 0)
    plsc.subcore_barrier()

    orow0 = pl.multiple_of(sid * _OUTR, 8)
    pltpu.sync_copy(s_sh.at[pl.ds(orow0, _OUTR)],
                    s_out.at[cid, pl.ds(orow0, _OUTR)])

    @pl.when(sid == _NS - 1)
    def _tail():
        tail = _NS * _OUTR
        pltpu.sync_copy(s_sh.at[pl.ds(tail, _N - tail)],
                        s_out.at[cid, pl.ds(tail, _N - tail)])


def _make_seg():
    out_type = [jax.ShapeDtypeStruct((_NC, _N, 2 * _D), jnp.float32)]
    scratch = [
        pltpu.VMEM((_IB, _CHUNK), jnp.int32),
        pltpu.VMEM((_IB, _CHUNK), jnp.int32),
        pltpu.VMEM((_CHUNK, 2 * _D), jnp.float32),
        pltpu.VMEM((_CHUNK, 2 * _D), jnp.float32),
        pltpu.VMEM((_CHUNK // 2, 2 * _D), jnp.float32),
        pltpu.VMEM((_ZR, 2 * _D), jnp.float32),
        pltpu.VMEM((_LANE,), jnp.float32),
        pltpu.VMEM_SHARED((_N, 2 * _D), jnp.float32),
    ]
    scratch += [pltpu.SemaphoreType.DMA] * 3
    mesh = plsc.VectorSubcoreMesh(core_axis_name="c", subcore_axis_name="s")
    return pl.kernel(_seg_body, out_type=out_type,
                     scratch_types=scratch, mesh=mesh)


_SEG = _make_seg()


def kernel(x, edge_index, edge_attr, params):
    f32 = jnp.float32
    row = lambda b: jnp.asarray(b, f32).reshape(1, -1)
    sca = lambda a: jnp.asarray(a, f32).reshape(1, 1)

    src3 = edge_index[0].astype(jnp.int32).reshape(_NW, _NBLK, _IB, _CHUNK)
    dst3 = edge_index[1].astype(jnp.int32).reshape(_NW, _NBLK, _IB, _CHUNK)

    layers = params["layers"]
    em = [lp["edge_mlp"] for lp in layers]
    nm = [lp["node_mlp"] for lp in layers]
    w1ij = [jnp.concatenate([p["W1"][:_D], p["W1"][_D:2 * _D]], axis=1)
            for p in em]
    bij = [jnp.concatenate([jnp.asarray(p["b1"], f32),
                            jnp.zeros((_D,), f32)]).reshape(1, 2 * _D)
           for p in em]
    ne = params["node_enc"]
    nd = jax.ShapeDtypeStruct((_N, _D), f32)
    td = jax.ShapeDtypeStruct((_N, 2 * _D), f32)
    h, t = pl.pallas_call(
        _node_enc_body, out_shape=[nd, td],
    )(x, ne["W1"], row(ne["b1"]), sca(ne["a1"]), ne["W2"], row(ne["b2"]),
      w1ij[0], bij[0])

    ee = params["edge_enc"]
    bd = lambda w: jnp.block([[w, jnp.zeros(w.shape, f32)],
                              [jnp.zeros(w.shape, f32), w]])
    dup = lambda b: jnp.concatenate([jnp.asarray(b, f32)] * 2).reshape(1, -1)
    ea2 = edge_attr.astype(f32).reshape(_E // 2, 8)
    wstd = jnp.stack([bd(p["W1"][2 * _D:]) for p in em])
    full2 = lambda i: (0, 0)
    ecs = pl.pallas_call(
        _edge_enc_body,
        grid=(_E // 2 // _BE2,),
        in_specs=[
            pl.BlockSpec((_BE2, 8), lambda i: (i, 0)),
            pl.BlockSpec((8, 2 * _D), full2),
            pl.BlockSpec((1, 2 * _D), full2),
            pl.BlockSpec((1, 1), full2),
            pl.BlockSpec((2 * _D, 2 * _D), full2),
            pl.BlockSpec((1, 2 * _D), full2),
            pl.BlockSpec((_L, 2 * _D, 2 * _D), lambda i: (0, 0, 0)),
        ],
        out_specs=[pl.BlockSpec((_BE2, 2 * _D), lambda i: (i, 0))] * _L,
        out_shape=[jax.ShapeDtypeStruct((_E // 2, 2 * _D), f32)] * _L,
    )(ea2, bd(ee["W1"]), dup(ee["b1"]), sca(ee["a1"]), bd(ee["W2"]),
      dup(ee["b2"]), wstd)

    for l in range(_L):
        a16 = jnp.full((_LANE,), em[l]["a1"], f32)
        (s_part,) = _SEG(dst3, src3, t, ecs[l], a16)

        common = (h, s_part, em[l]["W2"], row(em[l]["b2"]),
                  nm[l]["W1"][:_D], nm[l]["W1"][_D:], row(nm[l]["b1"]),
                  sca(nm[l]["a1"]), nm[l]["W2"], row(nm[l]["b2"]))
        if l < _L - 1:
            h, t = pl.pallas_call(
                _update_body, out_shape=[nd, td],
            )(*common, w1ij[l + 1], bij[l + 1])
        else:
            dec = params["dec"]
            out = pl.pallas_call(
                _final_body, out_shape=jax.ShapeDtypeStruct((_N, 3), f32),
            )(*common, dec["W1"], row(dec["b1"]), sca(dec["a1"]),
              dec["W2"], row(dec["b2"]))
    return out


# speedup vs baseline: 4.5507x; 1.5780x over previous
import functools

import jax
import jax.numpy as jnp
from jax import lax
from jax.experimental import pallas as pl
from jax.experimental.pallas import tpu as pltpu
from jax.experimental.pallas import tpu_sc as plsc

_N = 10000
_E = 320000
_D = 64
_L = 5

_NC = 2
_NS = 16
_LANE = 16
_NW = _NC * _NS
_EPW = _E // _NW
_CHUNK = 40
_NCHUNK = _EPW // _CHUNK
_IB = 25
_NBLK = _NCHUNK // _IB
_RPT = _N // _NS
_ZR = 25
_OUTR = 624
_BE = 4000


def _prelu(v, a):
    return jnp.where(v >= 0, v, a * v)


def _dot(a, b):
    return jnp.dot(a, b, preferred_element_type=jnp.float32)


def _node_enc_body(x_ref, w1, b1, a1, w2, b2, w1ij, be,
                   h_ref, t_ref):
    m = _prelu(_dot(x_ref[...], w1[...]) + b1[...], a1[0, 0])
    h = _dot(m, w2[...]) + b2[...]
    h_ref[...] = h
    t_ref[...] = _dot(h, w1ij[...]) + be[...]


def _edge_enc_body(ea_ref, w1, b1, a1, w2, b2, wst, o0, o1, o2, o3, o4):
    m = _prelu(_dot(ea_ref[...], w1[...]) + b1[...], a1[0, 0])
    e = _dot(m, w2[...]) + b2[...]
    for l, o in enumerate((o0, o1, o2, o3, o4)):
        o[...] = _dot(e, wst[l])


def _update_core(h_ref, s_ref, w2, b2, v1h, v1a, c1, an, v2, c2):
    h = h_ref[...]
    ssum = s_ref[0] + s_ref[1]
    s = ssum[:, :_D]
    deg = ssum[:, _D:_D + 1]
    aggr = _dot(s, w2[...]) + deg * b2[...]
    z = _prelu(_dot(h, v1h[...]) + _dot(aggr, v1a[...]) + c1[...], an[0, 0])
    return h + _dot(z, v2[...]) + c2[...]


def _update_body(h_ref, s_ref, w2, b2, v1h, v1a, c1, an, v2, c2,
                 nw1ij, nbe, hn_ref, t_ref):
    hn = _update_core(h_ref, s_ref, w2, b2, v1h, v1a, c1, an, v2, c2)
    hn_ref[...] = hn
    t_ref[...] = _dot(hn, nw1ij[...]) + nbe[...]


def _final_body(h_ref, s_ref, w2, b2, v1h, v1a, c1, an, v2, c2,
                dw1, db1, da, dw2, db2, out_ref):
    hn = _update_core(h_ref, s_ref, w2, b2, v1h, v1a, c1, an, v2, c2)
    t = _prelu(_dot(hn, dw1[...]) + db1[...], da[0, 0])
    out_ref[...] = _dot(t, dw2[...]) + db2[...]


def _seg_body(dst_hbm, src_hbm, t_hbm, ec_hbm, a_hbm, s_out,
              idxd, idxs, ti_v, tj_v, ec_v, z_v, a_v,
              s_sh, se0, si0, sj0, se1, si1, sj1):
    cid = lax.axis_index("c")
    sid = lax.axis_index("s")
    wid = sid * _NC + cid

    pltpu.sync_copy(a_hbm, a_v)
    alpha = a_v[...]

    zero = jnp.zeros((_LANE,), jnp.float32)
    one = jnp.ones((_LANE,), jnp.float32)

    def zrow(r, _):
        for c4 in range(2 * _D // _LANE):
            z_v[r, pl.ds(c4 * _LANE, _LANE)] = zero
        return 0

    lax.fori_loop(0, _ZR, zrow, 0)
    for j in range(_RPT // _ZR):
        pltpu.sync_copy(z_v, s_sh.at[pl.ds(sid * _RPT + j * _ZR, _ZR)])

    plsc.subcore_barrier()

    ksems = ((se0, si0, sj0), (se1, si1, sj1))

    def issue(b, u, ks):
        sem_e, sem_i, sem_j = ksems[ks]
        c = b * _IB + u
        base = pl.multiple_of(wid * _EPW + c * _CHUNK, _CHUNK)
        cp_e = pltpu.async_copy(
            ec_hbm.at[pl.ds(base, _CHUNK)], ec_v.at[ks], sem_e)
        cp_i = pltpu.async_copy(t_hbm.at[idxd.at[u]], ti_v.at[ks], sem_i)
        cp_j = pltpu.async_copy(t_hbm.at[idxs.at[u]], tj_v.at[ks], sem_j)
        return cp_e, cp_i, cp_j

    def blk(b, _):
        pltpu.sync_copy(dst_hbm.at[wid, b], idxd)
        pltpu.sync_copy(src_hbm.at[wid, b], idxs)
        cps = issue(b, 0, 0)
        for u in range(_IB):
            ks = u % 2
            for cp in cps:
                cp.wait()
            if u + 1 < _IB:
                cps = issue(b, u + 1, 1 - ks)

            def cw(r, _):
                for c4 in range(_D // _LANE):
                    sl = pl.ds(c4 * _LANE, _LANE)
                    v = (ti_v[ks, r, sl]
                         + tj_v[ks, r, pl.ds(_D + c4 * _LANE, _LANE)]
                         + ec_v[ks, r, sl])
                    ti_v[ks, r, sl] = jnp.where(v >= 0, v, alpha * v)
                for c4 in range(_D // _LANE, 2 * _D // _LANE):
                    ti_v[ks, r, pl.ds(c4 * _LANE, _LANE)] = (
                        one if c4 == _D // _LANE else zero)
                return 0

            lax.fori_loop(0, _CHUNK, cw, 0)
            pltpu.sync_copy(ti_v.at[ks], s_sh.at[idxd.at[u]], add=True)
        return 0

    lax.fori_loop(0, _NBLK, blk, 0)
    plsc.subcore_barrier()

    orow0 = pl.multiple_of(sid * _OUTR, 8)
    pltpu.sync_copy(s_sh.at[pl.ds(orow0, _OUTR)],
                    s_out.at[cid, pl.ds(orow0, _OUTR)])

    @pl.when(sid == _NS - 1)
    def _tail():
        tail = _NS * _OUTR
        pltpu.sync_copy(s_sh.at[pl.ds(tail, _N - tail)],
                        s_out.at[cid, pl.ds(tail, _N - tail)])


def _make_seg():
    out_type = [jax.ShapeDtypeStruct((_NC, _N, 2 * _D), jnp.float32)]
    scratch = [
        pltpu.VMEM((_IB, _CHUNK), jnp.int32),
        pltpu.VMEM((_IB, _CHUNK), jnp.int32),
        pltpu.VMEM((2, _CHUNK, 2 * _D), jnp.float32),
        pltpu.VMEM((2, _CHUNK, 2 * _D), jnp.float32),
        pltpu.VMEM((2, _CHUNK, _D), jnp.float32),
        pltpu.VMEM((_ZR, 2 * _D), jnp.float32),
        pltpu.VMEM((_LANE,), jnp.float32),
        pltpu.VMEM_SHARED((_N, 2 * _D), jnp.float32),
    ]
    scratch += [pltpu.SemaphoreType.DMA] * 6
    mesh = plsc.VectorSubcoreMesh(core_axis_name="c", subcore_axis_name="s")
    return pl.kernel(_seg_body, out_type=out_type,
                     scratch_types=scratch, mesh=mesh)


_SEG = _make_seg()


def kernel(x, edge_index, edge_attr, params):
    f32 = jnp.float32
    row = lambda b: jnp.asarray(b, f32).reshape(1, -1)
    sca = lambda a: jnp.asarray(a, f32).reshape(1, 1)

    src3 = edge_index[0].astype(jnp.int32).reshape(_NW, _NBLK, _IB, _CHUNK)
    dst3 = edge_index[1].astype(jnp.int32).reshape(_NW, _NBLK, _IB, _CHUNK)

    layers = params["layers"]
    em = [lp["edge_mlp"] for lp in layers]
    nm = [lp["node_mlp"] for lp in layers]
    w1ij = [jnp.concatenate([p["W1"][:_D], p["W1"][_D:2 * _D]], axis=1)
            for p in em]
    bij = [jnp.concatenate([jnp.asarray(p["b1"], f32),
                            jnp.zeros((_D,), f32)]).reshape(1, 2 * _D)
           for p in em]
    ne = params["node_enc"]
    nd = jax.ShapeDtypeStruct((_N, _D), f32)
    td = jax.ShapeDtypeStruct((_N, 2 * _D), f32)
    h, t = pl.pallas_call(
        _node_enc_body, out_shape=[nd, td],
    )(x, ne["W1"], row(ne["b1"]), sca(ne["a1"]), ne["W2"], row(ne["b2"]),
      w1ij[0], bij[0])

    ee = params["edge_enc"]
    wst = jnp.stack([p["W1"][2 * _D:] for p in em])
    full2 = lambda i: (0, 0)
    ecs = pl.pallas_call(
        _edge_enc_body,
        grid=(_E // _BE,),
        in_specs=[
            pl.BlockSpec((_BE, 4), lambda i: (i, 0)),
            pl.BlockSpec((4, _D), full2),
            pl.BlockSpec((1, _D), full2),
            pl.BlockSpec((1, 1), full2),
            pl.BlockSpec((_D, _D), full2),
            pl.BlockSpec((1, _D), full2),
            pl.BlockSpec((_L, _D, _D), lambda i: (0, 0, 0)),
        ],
        out_specs=[pl.BlockSpec((_BE, _D), lambda i: (i, 0))] * _L,
        out_shape=[jax.ShapeDtypeStruct((_E, _D), f32)] * _L,
    )(edge_attr.astype(f32), ee["W1"], row(ee["b1"]), sca(ee["a1"]),
      ee["W2"], row(ee["b2"]), wst)

    for l in range(_L):
        a16 = jnp.full((_LANE,), em[l]["a1"], f32)
        (s_part,) = _SEG(dst3, src3, t, ecs[l], a16)

        common = (h, s_part, em[l]["W2"], row(em[l]["b2"]),
                  nm[l]["W1"][:_D], nm[l]["W1"][_D:], row(nm[l]["b1"]),
                  sca(nm[l]["a1"]), nm[l]["W2"], row(nm[l]["b2"]))
        if l < _L - 1:
            h, t = pl.pallas_call(
                _update_body, out_shape=[nd, td],
            )(*common, w1ij[l + 1], bij[l + 1])
        else:
            dec = params["dec"]
            out = pl.pallas_call(
                _final_body, out_shape=jax.ShapeDtypeStruct((_N, 3), f32),
            )(*common, dec["W1"], row(dec["b1"]), sca(dec["a1"]),
              dec["W2"], row(dec["b2"]))
    return out


# speedup vs baseline: 5.3992x; 1.1864x over previous
import functools

import jax
import jax.numpy as jnp
from jax import lax
from jax.experimental import pallas as pl
from jax.experimental.pallas import tpu as pltpu
from jax.experimental.pallas import tpu_sc as plsc

_N = 10000
_E = 320000
_D = 64
_L = 5

_NC = 2
_NS = 16
_LANE = 16
_NW = _NC * _NS
_EPW = _E // _NW
_CHUNK = 40
_NCHUNK = _EPW // _CHUNK
_IB = 50
_NBLK = _NCHUNK // _IB
_RPT = _N // _NS
_ZR = 25
_OUTR = 624
_BE = 4000


def _prelu(v, a):
    return jnp.where(v >= 0, v, a * v)


def _dot(a, b):
    return jnp.dot(a, b, preferred_element_type=jnp.float32)


def _node_enc_body(x_ref, w1, b1, a1, w2, b2, w1ij, be,
                   h_ref, t_ref):
    m = _prelu(_dot(x_ref[...], w1[...]) + b1[...], a1[0, 0])
    h = _dot(m, w2[...]) + b2[...]
    h_ref[...] = h
    t_ref[...] = _dot(h, w1ij[...]) + be[...]


def _edge_enc_body(ea_ref, w1, b1, a1, w2, b2, wst, o0, o1, o2, o3, o4):
    m = _prelu(_dot(ea_ref[...], w1[...]) + b1[...], a1[0, 0])
    e = _dot(m, w2[...]) + b2[...]
    for l, o in enumerate((o0, o1, o2, o3, o4)):
        o[...] = _dot(e, wst[l])


def _update_core(h_ref, s_ref, w2, b2, v1h, v1a, c1, an, v2, c2):
    h = h_ref[...]
    ssum = s_ref[0] + s_ref[1]
    s = ssum[:, :_D]
    deg = ssum[:, _D:_D + 1]
    aggr = _dot(s, w2[...]) + deg * b2[...]
    z = _prelu(_dot(h, v1h[...]) + _dot(aggr, v1a[...]) + c1[...], an[0, 0])
    return h + _dot(z, v2[...]) + c2[...]


def _update_body(h_ref, s_ref, w2, b2, v1h, v1a, c1, an, v2, c2,
                 nw1ij, nbe, hn_ref, t_ref):
    hn = _update_core(h_ref, s_ref, w2, b2, v1h, v1a, c1, an, v2, c2)
    hn_ref[...] = hn
    t_ref[...] = _dot(hn, nw1ij[...]) + nbe[...]


def _final_body(h_ref, s_ref, w2, b2, v1h, v1a, c1, an, v2, c2,
                dw1, db1, da, dw2, db2, out_ref):
    hn = _update_core(h_ref, s_ref, w2, b2, v1h, v1a, c1, an, v2, c2)
    t = _prelu(_dot(hn, dw1[...]) + db1[...], da[0, 0])
    out_ref[...] = _dot(t, dw2[...]) + db2[...]


def _seg_body(dst_hbm, src_hbm, t_hbm, ec_hbm, a_hbm, s_out,
              idxd, idxs, ti_v, tj_v, ec_v, msg_v,
              s_sh, se0, si0, sj0, se1, si1, sj1):
    cid = lax.axis_index("c")
    sid = lax.axis_index("s")
    wid = sid * _NC + cid

    pltpu.sync_copy(a_hbm, msg_v.at[0, pl.ds(0, _LANE)])
    alpha = msg_v[0, pl.ds(0, _LANE)]

    zero = jnp.zeros((_LANE,), jnp.float32)
    one = jnp.ones((_LANE,), jnp.float32)

    def zrow(r, _):
        for c4 in range(2 * _D // _LANE):
            msg_v[r, pl.ds(c4 * _LANE, _LANE)] = zero
        return 0

    lax.fori_loop(0, 2 * _CHUNK, zrow, 0)
    nfull = _RPT // (2 * _CHUNK)
    for j in range(nfull):
        pltpu.sync_copy(msg_v,
                        s_sh.at[pl.ds(sid * _RPT + j * 2 * _CHUNK,
                                      2 * _CHUNK)])
    rem = _RPT - nfull * 2 * _CHUNK
    if rem:
        pltpu.sync_copy(msg_v.at[pl.ds(0, rem)],
                        s_sh.at[pl.ds(sid * _RPT + nfull * 2 * _CHUNK, rem)])

    def orow(r, _):
        msg_v[r, pl.ds(_D, _LANE)] = one
        return 0

    lax.fori_loop(0, 2 * _CHUNK, orow, 0)

    plsc.subcore_barrier()

    ksems = ((se0, si0, sj0), (se1, si1, sj1))

    def issue(b, c, ks):
        sem_e, sem_i, sem_j = ksems[ks]
        base = pl.multiple_of(wid * _EPW + (b * _IB + c) * _CHUNK, _CHUNK)
        idx = idxd.at[c // 2, pl.ds(ks * _CHUNK, _CHUNK)]
        idxj = idxs.at[c // 2, pl.ds(ks * _CHUNK, _CHUNK)]
        pltpu.async_copy(ec_hbm.at[pl.ds(base, _CHUNK)], ec_v.at[ks], sem_e)
        pltpu.async_copy(t_hbm.at[idx], ti_v.at[ks], sem_i)
        pltpu.async_copy(t_hbm.at[idxj], tj_v.at[ks], sem_j)

    def waitc(ks):
        sem_e, sem_i, sem_j = ksems[ks]
        pltpu.make_async_copy(ec_hbm.at[pl.ds(0, _CHUNK)], ec_v.at[ks],
                              sem_e).wait()
        pltpu.make_async_copy(t_hbm.at[pl.ds(0, _CHUNK)], ti_v.at[ks],
                              sem_i).wait()
        pltpu.make_async_copy(t_hbm.at[pl.ds(0, _CHUNK)], tj_v.at[ks],
                              sem_j).wait()

    def compute(ks):
        def rows(r8, _):
            for i in range(8):
                r = r8 * 8 + i
                for g in range(_D // _LANE):
                    sl = pl.ds(g * _LANE, _LANE)
                    v = (ti_v[ks, r, sl]
                         + tj_v[ks, r, pl.ds(_D + g * _LANE, _LANE)]
                         + ec_v[ks, r, sl])
                    msg_v[ks * _CHUNK + r, sl] = jnp.where(v >= 0, v,
                                                           alpha * v)
            return 0

        lax.fori_loop(0, _CHUNK // 8, rows, 0)

    def blk(b, _):
        pltpu.sync_copy(dst_hbm.at[wid, b], idxd)
        pltpu.sync_copy(src_hbm.at[wid, b], idxs)
        issue(b, 0, 0)
        issue(b, 1, 1)

        def pair(u2, _):
            for ks in range(2):
                c = 2 * u2 + ks
                waitc(ks)
                compute(ks)

                @pl.when(c + 2 < _IB)
                def _():
                    issue(b, c + 2, ks)

            pltpu.sync_copy(msg_v, s_sh.at[idxd.at[u2]], add=True)
            return 0

        lax.fori_loop(0, _IB // 2, pair, 0)
        return 0

    lax.fori_loop(0, _NBLK, blk, 0)
    plsc.subcore_barrier()

    orow0 = pl.multiple_of(sid * _OUTR, 8)
    pltpu.sync_copy(s_sh.at[pl.ds(orow0, _OUTR)],
                    s_out.at[cid, pl.ds(orow0, _OUTR)])

    @pl.when(sid == _NS - 1)
    def _tail():
        tail = _NS * _OUTR
        pltpu.sync_copy(s_sh.at[pl.ds(tail, _N - tail)],
                        s_out.at[cid, pl.ds(tail, _N - tail)])


def _make_seg():
    out_type = [jax.ShapeDtypeStruct((_NC, _N, 2 * _D), jnp.float32)]
    scratch = [
        pltpu.VMEM((_IB // 2, 2 * _CHUNK), jnp.int32),
        pltpu.VMEM((_IB // 2, 2 * _CHUNK), jnp.int32),
        pltpu.VMEM((2, _CHUNK, 2 * _D), jnp.float32),
        pltpu.VMEM((2, _CHUNK, 2 * _D), jnp.float32),
        pltpu.VMEM((2, _CHUNK, _D), jnp.float32),
        pltpu.VMEM((2 * _CHUNK, 2 * _D), jnp.float32),
        pltpu.VMEM_SHARED((_N, 2 * _D), jnp.float32),
    ]
    scratch += [pltpu.SemaphoreType.DMA] * 6
    mesh = plsc.VectorSubcoreMesh(core_axis_name="c", subcore_axis_name="s")
    return pl.kernel(_seg_body, out_type=out_type,
                     scratch_types=scratch, mesh=mesh)


_SEG = _make_seg()


def kernel(x, edge_index, edge_attr, params):
    f32 = jnp.float32
    row = lambda b: jnp.asarray(b, f32).reshape(1, -1)
    sca = lambda a: jnp.asarray(a, f32).reshape(1, 1)

    src3 = edge_index[0].astype(jnp.int32).reshape(
        _NW, _NBLK, _IB // 2, 2 * _CHUNK)
    dst3 = edge_index[1].astype(jnp.int32).reshape(
        _NW, _NBLK, _IB // 2, 2 * _CHUNK)

    layers = params["layers"]
    em = [lp["edge_mlp"] for lp in layers]
    nm = [lp["node_mlp"] for lp in layers]
    w1ij = [jnp.concatenate([p["W1"][:_D], p["W1"][_D:2 * _D]], axis=1)
            for p in em]
    bij = [jnp.concatenate([jnp.asarray(p["b1"], f32),
                            jnp.zeros((_D,), f32)]).reshape(1, 2 * _D)
           for p in em]
    ne = params["node_enc"]
    nd = jax.ShapeDtypeStruct((_N, _D), f32)
    td = jax.ShapeDtypeStruct((_N, 2 * _D), f32)
    h, t = pl.pallas_call(
        _node_enc_body, out_shape=[nd, td],
    )(x, ne["W1"], row(ne["b1"]), sca(ne["a1"]), ne["W2"], row(ne["b2"]),
      w1ij[0], bij[0])

    ee = params["edge_enc"]
    wst = jnp.stack([p["W1"][2 * _D:] for p in em])
    full2 = lambda i: (0, 0)
    ecs = pl.pallas_call(
        _edge_enc_body,
        grid=(_E // _BE,),
        in_specs=[
            pl.BlockSpec((_BE, 4), lambda i: (i, 0)),
            pl.BlockSpec((4, _D), full2),
            pl.BlockSpec((1, _D), full2),
            pl.BlockSpec((1, 1), full2),
            pl.BlockSpec((_D, _D), full2),
            pl.BlockSpec((1, _D), full2),
            pl.BlockSpec((_L, _D, _D), lambda i: (0, 0, 0)),
        ],
        out_specs=[pl.BlockSpec((_BE, _D), lambda i: (i, 0))] * _L,
        out_shape=[jax.ShapeDtypeStruct((_E, _D), f32)] * _L,
    )(edge_attr.astype(f32), ee["W1"], row(ee["b1"]), sca(ee["a1"]),
      ee["W2"], row(ee["b2"]), wst)

    for l in range(_L):
        a16 = jnp.full((_LANE,), em[l]["a1"], f32)
        (s_part,) = _SEG(dst3, src3, t, ecs[l], a16)

        common = (h, s_part, em[l]["W2"], row(em[l]["b2"]),
                  nm[l]["W1"][:_D], nm[l]["W1"][_D:], row(nm[l]["b1"]),
                  sca(nm[l]["a1"]), nm[l]["W2"], row(nm[l]["b2"]))
        if l < _L - 1:
            h, t = pl.pallas_call(
                _update_body, out_shape=[nd, td],
            )(*common, w1ij[l + 1], bij[l + 1])
        else:
            dec = params["dec"]
            out = pl.pallas_call(
                _final_body, out_shape=jax.ShapeDtypeStruct((_N, 3), f32),
            )(*common, dec["W1"], row(dec["b1"]), sca(dec["a1"]),
              dec["W2"], row(dec["b2"]))
    return out
